# baseline (speedup 1.0000x reference)
import functools

import jax
import jax.numpy as jnp
from jax import lax
from jax.experimental import pallas as pl
from jax.experimental.pallas import tpu as pltpu
from jax.experimental.pallas import tpu_sc as plsc

_LANES = 16
_NBUF = 5


@functools.lru_cache(maxsize=None)
def _emb_call(B, S, V, D):
    info = plsc.get_sparse_core_info()
    NC, NS = info.num_cores, info.num_subcores
    NW = NC * NS
    assert S % NW == 0
    s_per_w = S // NW
    SP = min(16, s_per_w)
    assert s_per_w % SP == 0 and D % _LANES == 0
    n_chunks = s_per_w // SP
    NU = n_chunks * B
    mesh = plsc.VectorSubcoreMesh(core_axis_name="c", subcore_axis_name="s")

    @functools.partial(
        pl.kernel,
        mesh=mesh,
        out_type=jax.ShapeDtypeStruct((B * S, D), jnp.float32),
        scratch_types=[
            pltpu.VMEM((B * s_per_w,), jnp.int32),
        ] + [pltpu.VMEM((SP, D), jnp.float32) for _ in range(_NBUF + 2)] + [
            pltpu.SemaphoreType.DMA for _ in range(2 * _NBUF + 3)
        ],
    )
    def emb(x_hbm, tok_hbm, pos_hbm, out_hbm, idx_all, *rest):
        toks = list(rest[:_NBUF])
        poss = list(rest[_NBUF:_NBUF + 2])
        sgs = list(rest[_NBUF + 2:2 * _NBUF + 2])
        sss = list(rest[2 * _NBUF + 2:3 * _NBUF + 2])
        sps = list(rest[3 * _NBUF + 2:3 * _NBUF + 4])
        si = rest[3 * _NBUF + 4]
        wid = lax.axis_index("s") * NC + lax.axis_index("c")
        s0 = wid * s_per_w
        units = [(ci, b) for ci in range(n_chunks) for b in range(B)]

        idx_descs = [
            pltpu.async_copy(x_hbm.at[b, pl.ds(s0, s_per_w)],
                             idx_all.at[pl.ds(b * s_per_w, s_per_w)], si)
            for b in range(B)
        ]

        def start_gather(u):
            ci, b = units[u]
            idx_ref = idx_all.at[pl.ds(b * s_per_w + ci * SP, SP)]
            return pltpu.async_copy(tok_hbm.at[idx_ref], toks[u % _NBUF],
                                    sgs[u % _NBUF])

        def start_pos(ci):
            return pltpu.async_copy(pos_hbm.at[pl.ds(s0 + ci * SP, SP)],
                                    poss[ci % 2], sps[ci % 2])

        pos_descs = {0: start_pos(0)}
        g_descs = {}
        for u in range(3):
            idx_descs[u].wait()
            g_descs[u] = start_gather(u)
        idx_descs[3].wait()
        s_descs = {}
        for u in range(NU):
            ci, b = units[u]
            slot = u % _NBUF
            if b == 0 and ci + 1 < n_chunks:
                pos_descs[ci + 1] = start_pos(ci + 1)
            if u + 3 < NU:
                if u + 3 - _NBUF in s_descs:
                    s_descs.pop(u + 3 - _NBUF).wait()
                g_descs[u + 3] = start_gather(u + 3)
            g_descs.pop(u).wait()
            if b == 0:
                pos_descs.pop(ci).wait()

            tok_v, pos_v = toks[slot], poss[ci % 2]

            def row_body(r, _):
                for c in range(D // _LANES):
                    sl = pl.ds(c * _LANES, _LANES)
                    plsc.addupdate(tok_v.at[r, sl], pos_v[r, sl])
                return 0

            lax.fori_loop(0, SP, row_body, 0)
            s_descs[u] = pltpu.async_copy(
                tok_v, out_hbm.at[pl.ds(b * S + s0 + ci * SP, SP)],
                sss[slot])
        for u in sorted(s_descs):
            s_descs.pop(u).wait()

    return emb


def kernel(x, tok_table, pos_table):
    B, S = x.shape
    V, D = tok_table.shape
    out = _emb_call(B, S, V, D)(x.astype(jnp.int32), tok_table, pos_table)
    return out.reshape(B, S, D)


# speedup vs baseline: 1.0085x; 1.0085x over previous
import functools

import jax
import jax.numpy as jnp
from jax import lax
from jax.experimental import pallas as pl
from jax.experimental.pallas import tpu as pltpu
from jax.experimental.pallas import tpu_sc as plsc

_LANES = 16
_NBUF = 5


@functools.lru_cache(maxsize=None)
def _emb_call(B, S, V, D):
    info = plsc.get_sparse_core_info()
    NC, NS = info.num_cores, info.num_subcores
    NW = NC * NS
    assert S % NW == 0
    s_per_w = S // NW
    SP = min(16, s_per_w)
    assert s_per_w % SP == 0 and D % _LANES == 0
    n_chunks = s_per_w // SP
    NU = n_chunks * B
    mesh = plsc.VectorSubcoreMesh(core_axis_name="c", subcore_axis_name="s")

    @functools.partial(
        pl.kernel,
        mesh=mesh,
        out_type=jax.ShapeDtypeStruct((B * S, D), jnp.float32),
        scratch_types=[
            pltpu.VMEM((B * s_per_w,), jnp.int32),
        ] + [pltpu.VMEM((SP, D), jnp.float32) for _ in range(_NBUF + 2)] + [
            pltpu.SemaphoreType.DMA for _ in range(2 * _NBUF + 3)
        ],
    )
    def emb(x_hbm, tok_hbm, pos_hbm, out_hbm, idx_all, *rest):
        toks = list(rest[:_NBUF])
        poss = list(rest[_NBUF:_NBUF + 2])
        sgs = list(rest[_NBUF + 2:2 * _NBUF + 2])
        sss = list(rest[2 * _NBUF + 2:3 * _NBUF + 2])
        sps = list(rest[3 * _NBUF + 2:3 * _NBUF + 4])
        si = rest[3 * _NBUF + 4]
        wid = lax.axis_index("s") * NC + lax.axis_index("c")
        s0 = wid * s_per_w
        units = [(ci, b) for ci in range(n_chunks) for b in range(B)]

        idx_descs = [
            pltpu.async_copy(x_hbm.at[b, pl.ds(s0, s_per_w)],
                             idx_all.at[pl.ds(b * s_per_w, s_per_w)], si)
            for b in range(B)
        ]

        def start_gather(u):
            ci, b = units[u]
            idx_ref = idx_all.at[pl.ds(b * s_per_w + ci * SP, SP)]
            return pltpu.async_copy(tok_hbm.at[idx_ref], toks[u % _NBUF],
                                    sgs[u % _NBUF])

        def start_pos(ci):
            return pltpu.async_copy(pos_hbm.at[pl.ds(s0 + ci * SP, SP)],
                                    poss[ci % 2], sps[ci % 2])

        pos_descs = {0: start_pos(0)}
        g_descs = {}
        idx_waited = set()
        for u in range(min(3, NU)):
            if units[u][1] not in idx_waited:
                idx_waited.add(units[u][1])
                idx_descs[units[u][1]].wait()
            g_descs[u] = start_gather(u)
        for b in range(B):
            if b not in idx_waited:
                idx_descs[b].wait()
        s_descs = {}
        for u in range(NU):
            ci, b = units[u]
            slot = u % _NBUF
            if b == 0 and ci + 1 < n_chunks:
                pos_descs[ci + 1] = start_pos(ci + 1)
            if u + 3 < NU:
                if u + 3 - _NBUF in s_descs:
                    s_descs.pop(u + 3 - _NBUF).wait()
                g_descs[u + 3] = start_gather(u + 3)
            g_descs.pop(u).wait()
            if b == 0:
                pos_descs.pop(ci).wait()

            tok_v, pos_v = toks[slot], poss[ci % 2]

            def row_body(r, _):
                for c in range(D // _LANES):
                    sl = pl.ds(c * _LANES, _LANES)
                    plsc.addupdate(tok_v.at[r, sl], pos_v[r, sl])
                return 0

            lax.fori_loop(0, SP, row_body, 0)
            s_descs[u] = pltpu.async_copy(
                tok_v, out_hbm.at[pl.ds(b * S + s0 + ci * SP, SP)],
                sss[slot])
        for u in sorted(s_descs):
            s_descs.pop(u).wait()

    return emb


def kernel(x, tok_table, pos_table):
    B, S = x.shape
    V, D = tok_table.shape
    out = _emb_call(B, S, V, D)(x.astype(jnp.int32), tok_table, pos_table)
    return out.reshape(B, S, D)
